# baseline (speedup 1.0000x reference)
import jax, jax.numpy as jnp
from jax.experimental import pallas as pl
from jax.experimental.pallas import tpu as pltpu
_BR = 4096
def _z(out_ref):
    out_ref[...] = jnp.zeros((_BR, 1000), jnp.float32)
def kernel(inputs):
    return pl.pallas_call(
        _z,
        grid=(16384 // _BR,),
        out_specs=pl.BlockSpec((_BR, 1000), lambda i: (i, 0)),
        out_shape=jax.ShapeDtypeStruct((16384, 1000), jnp.float32),
        compiler_params=pltpu.CompilerParams(dimension_semantics=("parallel",)),
    )()


# speedup vs baseline: 1.0203x; 1.0203x over previous
import jax
import jax.numpy as jnp
from jax.experimental import pallas as pl
from jax.experimental.pallas import tpu as pltpu

_N = 16384
_DEPTH = 1000
_BR = 1024
_STEPS = _N // _BR
_SLOTS = 4


def _copy(i, out_ref, scratch, sems):
    slot = jax.lax.rem(i, _SLOTS)
    return pltpu.make_async_copy(
        scratch.at[slot],
        out_ref.at[pl.ds(i * _BR, _BR), :],
        sems.at[slot],
    )


def _onehot_block(idx_ref, out_ref, scratch, sems):
    i = pl.program_id(0)
    slot = jax.lax.rem(i, _SLOTS)

    @pl.when(i >= _SLOTS)
    def _wait_prev():
        _copy(i - _SLOTS, out_ref, scratch, sems).wait()

    idx = idx_ref[pl.ds(i * _BR, _BR)].reshape(_BR, 1)
    cols = jax.lax.broadcasted_iota(jnp.int32, (_BR, _DEPTH), 1)
    scratch.at[slot][...] = (idx == cols).astype(jnp.float32)
    _copy(i, out_ref, scratch, sems).start()

    @pl.when(i == _STEPS - 1)
    def _drain():
        for k in range(_SLOTS):
            _copy(_STEPS - _SLOTS + k, out_ref, scratch, sems).wait()


def kernel(inputs):
    idx = inputs.astype(jnp.int32)
    return pl.pallas_call(
        _onehot_block,
        grid=(_STEPS,),
        in_specs=[pl.BlockSpec((_N,), lambda i: (0,))],
        out_specs=pl.BlockSpec(memory_space=pl.ANY),
        out_shape=jax.ShapeDtypeStruct((_N, _DEPTH), jnp.float32),
        scratch_shapes=[
            pltpu.VMEM((_SLOTS, _BR, _DEPTH), jnp.float32),
            pltpu.SemaphoreType.DMA((_SLOTS,)),
        ],
        compiler_params=pltpu.CompilerParams(
            dimension_semantics=("arbitrary",),
        ),
    )(idx)


# speedup vs baseline: 1.0270x; 1.0065x over previous
import jax
import jax.numpy as jnp
from jax.experimental import pallas as pl
from jax.experimental.pallas import tpu as pltpu

_N = 16384
_DEPTH = 1000
_BR = 1024


def _onehot_block(idx_ref, out_ref):
    idx = idx_ref[...].reshape(_BR, 1)
    cols = jax.lax.broadcasted_iota(jnp.int32, (_BR, _DEPTH), 1)
    out_ref[...] = (idx == cols).astype(jnp.float32)


def kernel(inputs):
    idx = inputs.astype(jnp.int32)
    return pl.pallas_call(
        _onehot_block,
        grid=(_N // _BR,),
        in_specs=[pl.BlockSpec((_BR,), lambda i: (i,))],
        out_specs=pl.BlockSpec((_BR, _DEPTH), lambda i: (i, 0)),
        out_shape=jax.ShapeDtypeStruct((_N, _DEPTH), jnp.float32),
        compiler_params=pltpu.CompilerParams(
            dimension_semantics=("arbitrary",),
        ),
    )(idx)
